# baseline (speedup 1.0000x reference)
import functools

import jax
import jax.numpy as jnp
from jax import lax
from jax.experimental import pallas as pl
from jax.experimental.pallas import tpu as pltpu
from jax.experimental.pallas import tpu_sc as plsc

B, L, W_CHARS = 1024, 50, 16
CHAR_SIZE = 128
CHAR_DIM = 30
CD_PAD = 32
HIDDEN = 1024
N_TOK = B * L
N_PAIR = W_CHARS // 2
N_LOOK = N_TOK * N_PAIR
PCD = 2 * CD_PAD
K_PAD = W_CHARS * CD_PAD

_NC, _NS = 2, 16
_NW = _NC * _NS
_LPW = N_LOOK // _NW
_GLOOK = 1280
_NG = _LPW // _GLOOK

_sc_mesh = plsc.VectorSubcoreMesh(
    core_axis_name="c", subcore_axis_name="s", num_cores=_NC, num_subcores=_NS
)


@functools.partial(
    pl.kernel,
    out_type=jax.ShapeDtypeStruct((N_LOOK, PCD), jnp.bfloat16),
    mesh=_sc_mesh,
    scratch_types=[
        pltpu.VMEM((_LPW,), jnp.int32),
        pltpu.VMEM((2, _GLOOK, PCD), jnp.bfloat16),
        pltpu.SemaphoreType.DMA,
        pltpu.SemaphoreType.DMA,
    ],
    compiler_params=pltpu.CompilerParams(use_tc_tiling_on_sc=False),
)
def _sc_gather(idx_hbm, tab_hbm, emb_hbm, idx_v, rows_v, gsem, wsem):
    wid = lax.axis_index("s") * _NC + lax.axis_index("c")
    pltpu.sync_copy(idx_hbm.at[wid], idx_v)
    base = wid * _LPW

    def write_desc(g, b):
        return pltpu.make_async_copy(
            rows_v.at[b],
            emb_hbm.at[pl.ds(base + g * _GLOOK, _GLOOK)],
            wsem,
        )

    def gather_desc(g, b):
        return pltpu.make_async_copy(
            tab_hbm.at[idx_v.at[pl.ds(g * _GLOOK, _GLOOK)]],
            rows_v.at[b],
            gsem,
        )

    @pl.loop(0, _NG, step=2)
    def _group(g0):
        for nb in range(2):
            g = g0 + nb

            @pl.when(g >= 2)
            def _():
                write_desc(g - 2, nb).wait()

            gather_desc(g, nb).start()
            gather_desc(g, nb).wait()
            write_desc(g, nb).start()

    for nb in range(2):
        write_desc(_NG - 2 + nb, nb).wait()


_TB = B


def _mm_body(e_ref, wt_ref, o_ref):
    e = e_ref[:].reshape(_TB, K_PAD)
    o_ref[0] = jnp.dot(e, wt_ref[:], preferred_element_type=jnp.float32)


@jax.jit
def kernel(X, char_table, W_proj):
    xt = X.transpose(1, 0, 2)
    idx = (xt[..., 0::2] * CHAR_SIZE + xt[..., 1::2]).reshape(_NW, _LPW)
    tab32 = jnp.pad(char_table, ((0, 0), (0, CD_PAD - CHAR_DIM))).astype(jnp.bfloat16)
    tabp = jnp.concatenate(
        [jnp.repeat(tab32, CHAR_SIZE, axis=0), jnp.tile(tab32, (CHAR_SIZE, 1))],
        axis=1,
    )
    emb = _sc_gather(idx, tabp)

    wt = jnp.pad(
        W_proj.reshape(HIDDEN, W_CHARS, CHAR_DIM),
        ((0, 0), (0, 0), (0, CD_PAD - CHAR_DIM)),
    ).reshape(HIDDEN, K_PAD).T.astype(jnp.bfloat16)

    out = pl.pallas_call(
        _mm_body,
        grid=(N_TOK // _TB,),
        in_specs=[
            pl.BlockSpec((_TB * 2, K_PAD // 2), lambda i: (i, 0)),
            pl.BlockSpec((K_PAD, HIDDEN), lambda i: (0, 0)),
        ],
        out_specs=pl.BlockSpec((1, B, HIDDEN), lambda i: (i, 0, 0)),
        out_shape=jax.ShapeDtypeStruct((L, B, HIDDEN), jnp.float32),
    )(emb.reshape(N_TOK * 2, K_PAD // 2), wt)
    return out.transpose(1, 0, 2)
